# Initial kernel scaffold; baseline (speedup 1.0000x reference)
import jax
import jax.numpy as jnp
from jax.experimental import pallas as pl


def kernel(h, t, r, ent_embeddings, rel_embeddings, normal_vectors):
    raise NotImplementedError("write your pallas kernel here")


# speedup vs baseline: 1.7192x; 1.7192x over previous
import functools

import jax
import jax.numpy as jnp
from jax import lax
from jax.experimental import pallas as pl
from jax.experimental.pallas import tpu as pltpu
from jax.experimental.pallas import tpu_sc as plsc

D = 128
B = 4096
NC = 2
NS = 16
L = 16
NW = NC * NS
BPW = B // NW
C = D // L

_mesh = plsc.VectorSubcoreMesh(core_axis_name="c", subcore_axis_name="s")


@functools.partial(
    pl.kernel,
    mesh=_mesh,
    out_type=jax.ShapeDtypeStruct((B,), jnp.float32),
    scratch_types=[
        pltpu.VMEM((BPW,), jnp.int32),
        pltpu.VMEM((BPW,), jnp.int32),
        pltpu.VMEM((BPW,), jnp.int32),
        pltpu.VMEM((BPW, D), jnp.float32),
        pltpu.VMEM((BPW, D), jnp.float32),
        pltpu.VMEM((BPW, D), jnp.float32),
        pltpu.VMEM((BPW, D), jnp.float32),
        pltpu.VMEM((BPW,), jnp.float32),
        pltpu.SemaphoreType.DMA,
    ],
)
def _transh_sc(h_hbm, t_hbm, r_hbm, ent_hbm, rel_hbm, nrm_hbm, out_hbm,
               hidx, tidx, ridx, hrows, trows, rrows, nrows, outv, sem):
    wid = lax.axis_index("s") * NC + lax.axis_index("c")
    base = wid * BPW

    pltpu.sync_copy(h_hbm.at[pl.ds(base, BPW)], hidx)
    pltpu.sync_copy(t_hbm.at[pl.ds(base, BPW)], tidx)
    pltpu.sync_copy(r_hbm.at[pl.ds(base, BPW)], ridx)

    cp1 = pltpu.async_copy(ent_hbm.at[hidx], hrows, sem)
    cp2 = pltpu.async_copy(ent_hbm.at[tidx], trows, sem)
    cp3 = pltpu.async_copy(rel_hbm.at[ridx], rrows, sem)
    cp4 = pltpu.async_copy(nrm_hbm.at[ridx], nrows, sem)
    cp1.wait()
    cp2.wait()
    cp3.wait()
    cp4.wait()

    lanes = lax.iota(jnp.int32, L)

    dnums = lax.GatherDimensionNumbers(
        offset_dims=(), collapsed_slice_dims=(0,), start_index_map=(0,))

    def permute(v, idx):
        return lax.gather(v, idx[:, None], dnums, (1,),
                          mode=lax.GatherScatterMode.PROMISE_IN_BOUNDS)

    def allreduce_sum(v):
        for k in (8, 4, 2, 1):
            v = v + permute(v, lanes ^ k)
        return v

    def body(g, carry):
        scores = jnp.zeros((L,), jnp.float32)
        for j in range(L):
            i = g * L + j
            dvs = []
            nvs = []
            dot = jnp.zeros((L,), jnp.float32)
            for c in range(C):
                hv = hrows[i, pl.ds(c * L, L)]
                tv = trows[i, pl.ds(c * L, L)]
                nv = nrows[i, pl.ds(c * L, L)]
                d = hv - tv
                dvs.append(d)
                nvs.append(nv)
                dot = dot + d * nv
            dots = allreduce_sum(dot)
            sacc = jnp.zeros((L,), jnp.float32)
            for c in range(C):
                rv = rrows[i, pl.ds(c * L, L)]
                s = dvs[c] + rv - dots * nvs[c]
                sacc = sacc + jnp.abs(s)
            scores = jnp.where(lanes == j, allreduce_sum(sacc), scores)
        outv[pl.ds(g * L, L)] = scores
        return carry

    lax.fori_loop(0, BPW // L, body, 0)
    pltpu.sync_copy(outv, out_hbm.at[pl.ds(base, BPW)])


def kernel(h, t, r, ent_embeddings, rel_embeddings, normal_vectors):
    return _transh_sc(
        h.astype(jnp.int32),
        t.astype(jnp.int32),
        r.astype(jnp.int32),
        ent_embeddings,
        rel_embeddings,
        normal_vectors,
    )
